# baseline (device time: 108154 ns/iter reference)
import jax
import jax.numpy as jnp
from jax import lax
from jax.experimental import pallas as pl
from jax.experimental.pallas import tpu as pltpu

CHUNKS_PER_B = 8
N_SLOTS = 4


def kernel(O, Wo):
    B, S, H, D = O.shape
    K = H * D
    N = Wo.shape[1]
    S_half = S // 2
    S_c = S_half // CHUNKS_PER_B
    n_chunks = B * CHUNKS_PER_B

    O2 = O.reshape(B, S, K)
    chunks = [(b, j * S_c) for b in range(B) for j in range(CHUNKS_PER_B)]

    def body(o_ref, w_ref, out_ref, o_slots, w_vmem, send_buf, recv_buf,
             o_sems, w_sem, send_sems, recv_sems):
        my_x = lax.axis_index("x")
        my_y = lax.axis_index("y")
        my_z = lax.axis_index("z")
        other_y = 1 - my_y
        neighbor = (my_x, other_y, my_z)

        w_cp = pltpu.make_async_copy(w_ref, w_vmem, w_sem)
        w_cp.start()

        nb = other_y * S_half
        mine = my_y * S_half

        jobs = [(b, nb + s0, s0) for b, s0 in chunks] + \
               [(b, mine + s0, s0) for b, s0 in chunks]
        cps = [None] * len(jobs)

        def start_copy(c):
            b, row0, _ = jobs[c]
            cp = pltpu.make_async_copy(
                o_ref.at[b, pl.ds(row0, S_c), :],
                o_slots.at[c % N_SLOTS],
                o_sems.at[c % N_SLOTS],
            )
            cp.start()
            cps[c] = cp

        for c in range(N_SLOTS):
            start_copy(c)

        barrier_sem = pltpu.get_barrier_semaphore()
        pl.semaphore_signal(
            barrier_sem, inc=1,
            device_id=neighbor, device_id_type=pl.DeviceIdType.MESH,
        )
        pl.semaphore_wait(barrier_sem, 1)

        w_cp.wait()

        rdmas = []
        for c, (b, row0, s0) in enumerate(jobs):
            cps[c].wait()
            mm = jnp.dot(
                o_slots[c % N_SLOTS], w_vmem[:, :],
                preferred_element_type=jnp.float32,
            )
            if c < n_chunks:
                send_buf[b, pl.ds(s0, S_c), :] = mm
            else:
                out_ref[b, pl.ds(s0, S_c), :] = mm
            if c + N_SLOTS < len(jobs):
                start_copy(c + N_SLOTS)
            if c < n_chunks:
                rdma = pltpu.make_async_remote_copy(
                    src_ref=send_buf.at[b, pl.ds(s0, S_c), :],
                    dst_ref=recv_buf.at[b, pl.ds(s0, S_c), :],
                    send_sem=send_sems.at[c],
                    recv_sem=recv_sems.at[c],
                    device_id=neighbor,
                    device_id_type=pl.DeviceIdType.MESH,
                )
                rdma.start()
                rdmas.append(rdma)

        for c, (b, s0) in enumerate(chunks):
            rdmas[c].wait_recv()
            out_ref[b, pl.ds(s0, S_c), :] += recv_buf[b, pl.ds(s0, S_c), :]

        for rdma in rdmas:
            rdma.wait_send()

    return pl.pallas_call(
        body,
        out_shape=jax.ShapeDtypeStruct((B, S_half, N), jnp.float32),
        in_specs=[
            pl.BlockSpec(memory_space=pl.ANY),
            pl.BlockSpec(memory_space=pl.ANY),
        ],
        out_specs=pl.BlockSpec(memory_space=pltpu.VMEM),
        scratch_shapes=[
            pltpu.VMEM((N_SLOTS, S_c, K), jnp.float32),
            pltpu.VMEM((K, N), jnp.float32),
            pltpu.VMEM((B, S_half, N), jnp.float32),
            pltpu.VMEM((B, S_half, N), jnp.float32),
            pltpu.SemaphoreType.DMA((N_SLOTS,)),
            pltpu.SemaphoreType.DMA,
            pltpu.SemaphoreType.DMA((n_chunks,)),
            pltpu.SemaphoreType.DMA((n_chunks,)),
        ],
        compiler_params=pltpu.CompilerParams(collective_id=0),
    )(O2, Wo)


# device time: 108114 ns/iter; 1.0004x vs baseline; 1.0004x over previous
import jax
import jax.numpy as jnp
from jax import lax
from jax.experimental import pallas as pl
from jax.experimental.pallas import tpu as pltpu

N_SLOTS = 4
_SIZES = {0: [64, 64, 128, 128, 128], 1: [128, 128, 128, 64, 64]}


def kernel(O, Wo):
    B, S, H, D = O.shape
    K = H * D
    N = Wo.shape[1]
    S_half = S // 2
    S_c = max(max(v) for v in _SIZES.values())

    O2 = O.reshape(B, S, K)
    chunks = []
    for b in range(B):
        s0 = 0
        for sz in _SIZES[b]:
            chunks.append((b, s0, sz))
            s0 += sz
        assert s0 == S_half
    n_chunks = len(chunks)

    def body(o_ref, w_ref, out_ref, o_slots, w_vmem, send_buf, recv_buf,
             o_sems, w_sem, send_sems, recv_sems):
        my_x = lax.axis_index("x")
        my_y = lax.axis_index("y")
        my_z = lax.axis_index("z")
        other_y = 1 - my_y
        neighbor = (my_x, other_y, my_z)

        w_cp = pltpu.make_async_copy(w_ref, w_vmem, w_sem)
        w_cp.start()

        nb = other_y * S_half
        mine = my_y * S_half

        jobs = [(b, nb + s0, s0, sz) for b, s0, sz in chunks] + \
               [(b, mine + s0, s0, sz) for b, s0, sz in chunks]
        cps = [None] * len(jobs)

        def start_copy(c):
            b, row0, _, sz = jobs[c]
            cp = pltpu.make_async_copy(
                o_ref.at[b, pl.ds(row0, sz), :],
                o_slots.at[c % N_SLOTS, pl.ds(0, sz), :],
                o_sems.at[c % N_SLOTS],
            )
            cp.start()
            cps[c] = cp

        for c in range(N_SLOTS):
            start_copy(c)

        barrier_sem = pltpu.get_barrier_semaphore()
        pl.semaphore_signal(
            barrier_sem, inc=1,
            device_id=neighbor, device_id_type=pl.DeviceIdType.MESH,
        )
        pl.semaphore_wait(barrier_sem, 1)

        w_cp.wait()

        rdmas = []
        for c, (b, row0, s0, sz) in enumerate(jobs):
            cps[c].wait()
            mm = jnp.dot(
                o_slots[c % N_SLOTS, pl.ds(0, sz), :], w_vmem[:, :],
                preferred_element_type=jnp.float32,
            )
            if c < n_chunks:
                send_buf[b, pl.ds(s0, sz), :] = mm
            else:
                out_ref[b, pl.ds(s0, sz), :] = mm
            if c + N_SLOTS < len(jobs):
                start_copy(c + N_SLOTS)
            if c < n_chunks:
                rdma = pltpu.make_async_remote_copy(
                    src_ref=send_buf.at[b, pl.ds(s0, sz), :],
                    dst_ref=recv_buf.at[b, pl.ds(s0, sz), :],
                    send_sem=send_sems.at[c],
                    recv_sem=recv_sems.at[c],
                    device_id=neighbor,
                    device_id_type=pl.DeviceIdType.MESH,
                )
                rdma.start()
                rdmas.append(rdma)

        for c, (b, s0, sz) in enumerate(chunks):
            rdmas[c].wait_recv()
            out_ref[b, pl.ds(s0, sz), :] += recv_buf[b, pl.ds(s0, sz), :]

        for rdma in rdmas:
            rdma.wait_send()

    return pl.pallas_call(
        body,
        out_shape=jax.ShapeDtypeStruct((B, S_half, N), jnp.float32),
        in_specs=[
            pl.BlockSpec(memory_space=pl.ANY),
            pl.BlockSpec(memory_space=pl.ANY),
        ],
        out_specs=pl.BlockSpec(memory_space=pltpu.VMEM),
        scratch_shapes=[
            pltpu.VMEM((N_SLOTS, S_c, K), jnp.float32),
            pltpu.VMEM((K, N), jnp.float32),
            pltpu.VMEM((B, S_half, N), jnp.float32),
            pltpu.VMEM((B, S_half, N), jnp.float32),
            pltpu.SemaphoreType.DMA((N_SLOTS,)),
            pltpu.SemaphoreType.DMA,
            pltpu.SemaphoreType.DMA((n_chunks,)),
            pltpu.SemaphoreType.DMA((n_chunks,)),
        ],
        compiler_params=pltpu.CompilerParams(collective_id=0),
    )(O2, Wo)


# device time: 108040 ns/iter; 1.0011x vs baseline; 1.0007x over previous
import jax
import jax.numpy as jnp
from jax import lax
from jax.experimental import pallas as pl
from jax.experimental.pallas import tpu as pltpu

N_SLOTS = 4
_SIZES = {0: [128, 128, 128, 128], 1: [128, 128, 128, 128]}


def kernel(O, Wo):
    B, S, H, D = O.shape
    K = H * D
    N = Wo.shape[1]
    S_half = S // 2
    S_c = max(max(v) for v in _SIZES.values())

    O2 = O.reshape(B, S, K)
    chunks = []
    for b in range(B):
        s0 = 0
        for sz in _SIZES[b]:
            chunks.append((b, s0, sz))
            s0 += sz
        assert s0 == S_half
    n_chunks = len(chunks)

    def body(o_ref, w_ref, out_ref, o_slots, w_vmem, send_buf, recv_buf,
             o_sems, w_sem, send_sems, recv_sems):
        my_x = lax.axis_index("x")
        my_y = lax.axis_index("y")
        my_z = lax.axis_index("z")
        other_y = 1 - my_y
        neighbor = (my_x, other_y, my_z)

        w_cp = pltpu.make_async_copy(w_ref, w_vmem, w_sem)
        w_cp.start()

        nb = other_y * S_half
        mine = my_y * S_half

        jobs = [(b, nb + s0, s0, sz) for b, s0, sz in chunks] + \
               [(b, mine + s0, s0, sz) for b, s0, sz in chunks]
        cps = [None] * len(jobs)

        def start_copy(c):
            b, row0, _, sz = jobs[c]
            cp = pltpu.make_async_copy(
                o_ref.at[b, pl.ds(row0, sz), :],
                o_slots.at[c % N_SLOTS, pl.ds(0, sz), :],
                o_sems.at[c % N_SLOTS],
            )
            cp.start()
            cps[c] = cp

        for c in range(N_SLOTS):
            start_copy(c)

        barrier_sem = pltpu.get_barrier_semaphore()
        pl.semaphore_signal(
            barrier_sem, inc=1,
            device_id=neighbor, device_id_type=pl.DeviceIdType.MESH,
        )
        pl.semaphore_wait(barrier_sem, 1)

        w_cp.wait()

        rdmas = []
        for c, (b, row0, s0, sz) in enumerate(jobs):
            cps[c].wait()
            mm = jnp.dot(
                o_slots[c % N_SLOTS, pl.ds(0, sz), :], w_vmem[:, :],
                preferred_element_type=jnp.float32,
            )
            if c < n_chunks:
                send_buf[b, pl.ds(s0, sz), :] = mm
            else:
                out_ref[b, pl.ds(s0, sz), :] = mm
            if c + N_SLOTS < len(jobs):
                start_copy(c + N_SLOTS)
            if c < n_chunks:
                rdma = pltpu.make_async_remote_copy(
                    src_ref=send_buf.at[b, pl.ds(s0, sz), :],
                    dst_ref=recv_buf.at[b, pl.ds(s0, sz), :],
                    send_sem=send_sems.at[c],
                    recv_sem=recv_sems.at[c],
                    device_id=neighbor,
                    device_id_type=pl.DeviceIdType.MESH,
                )
                rdma.start()
                rdmas.append(rdma)

        for c, (b, s0, sz) in enumerate(chunks):
            rdmas[c].wait_recv()
            out_ref[b, pl.ds(s0, sz), :] += recv_buf[b, pl.ds(s0, sz), :]

        for rdma in rdmas:
            rdma.wait_send()

    return pl.pallas_call(
        body,
        out_shape=jax.ShapeDtypeStruct((B, S_half, N), jnp.float32),
        in_specs=[
            pl.BlockSpec(memory_space=pl.ANY),
            pl.BlockSpec(memory_space=pl.ANY),
        ],
        out_specs=pl.BlockSpec(memory_space=pltpu.VMEM),
        scratch_shapes=[
            pltpu.VMEM((N_SLOTS, S_c, K), jnp.float32),
            pltpu.VMEM((K, N), jnp.float32),
            pltpu.VMEM((B, S_half, N), jnp.float32),
            pltpu.VMEM((B, S_half, N), jnp.float32),
            pltpu.SemaphoreType.DMA((N_SLOTS,)),
            pltpu.SemaphoreType.DMA,
            pltpu.SemaphoreType.DMA((n_chunks,)),
            pltpu.SemaphoreType.DMA((n_chunks,)),
        ],
        compiler_params=pltpu.CompilerParams(collective_id=0),
    )(O2, Wo)
